# Initial kernel scaffold; baseline (speedup 1.0000x reference)
import functools

import jax
import jax.numpy as jnp
from jax import lax
from jax.experimental import pallas as pl
from jax.experimental.pallas import tpu as pltpu
from jax.experimental.pallas import tpu_sc as plsc

_L = 16
_NS = 16
_NPAD = 5120
_NB = _NPAD // _L
_VPT = _NB // _NS
_T = 0.3
_ST = 0.05
_BIG = 2e30
_EPS = 1e-9


def _nms_body(y1h, x1h, y2h, x2h, sh, outh, y1, x1, y2, x2, s, keep, slot, shared):
    sid = lax.axis_index("s")

    pltpu.sync_copy(y1h, y1)
    pltpu.sync_copy(x1h, x1)
    pltpu.sync_copy(y2h, y2)
    pltpu.sync_copy(x2h, x2)
    pltpu.sync_copy(sh, s)

    def _init(b, _):
        off = b * _L
        sv = s[pl.ds(off, _L)]
        keep[pl.ds(off, _L)] = jnp.where(sv > _ST, 1.0, 0.0)
        return 0

    lax.fori_loop(0, _NB, _init, 0)

    lane = lax.iota(jnp.int32, _L)

    def _chunk(k, _):
        owner = lax.rem(k, _NS)
        base = k * _L

        @pl.when(sid == owner)
        def _():
            py1 = y1[pl.ds(base, _L)]
            px1 = x1[pl.ds(base, _L)]
            py2 = y2[pl.ds(base, _L)]
            px2 = x2[pl.ds(base, _L)]
            parea = (py2 - py1) * (px2 - px1)
            for l in range(_L):
                ay1 = y1[base + l]
                ax1 = x1[base + l]
                ay2 = y2[base + l]
                ax2 = x2[base + l]
                aarea = (ay2 - ay1) * (ax2 - ax1)
                kl = keep[base + l]
                hh = jnp.maximum(jnp.minimum(ay2, py2) - jnp.maximum(ay1, py1), 0.0)
                ww = jnp.maximum(jnp.minimum(ax2, px2) - jnp.maximum(ax1, px1), 0.0)
                inter = hh * ww
                iou = inter / (aarea + parea - inter + _EPS)
                thr = jnp.where(kl > 0.0, jnp.float32(_T), jnp.float32(_BIG))
                sup = (iou > thr) & (lane > l)
                kv = keep[pl.ds(base, _L)]
                keep[pl.ds(base, _L)] = jnp.where(sup, 0.0, kv)
            pltpu.sync_copy(keep.at[pl.ds(base, _L)], shared.at[pl.ds(base, _L)])

        plsc.subcore_barrier()

        pltpu.sync_copy(shared.at[pl.ds(base, _L)], slot)
        pys, pxs, pye, pxe, pa, pthr = [], [], [], [], [], []
        for p in range(_L):
            ay1 = y1[base + p]
            ax1 = x1[base + p]
            ay2 = y2[base + p]
            ax2 = x2[base + p]
            pys.append(ay1)
            pxs.append(ax1)
            pye.append(ay2)
            pxe.append(ax2)
            pa.append((ay2 - ay1) * (ax2 - ax1))
            pthr.append(jnp.where(slot[p] > 0.0, jnp.float32(_T), jnp.float32(_BIG)))

        jlo = lax.div(k - sid + _NS, _NS)

        def _apply(j, _):
            off = (j * _NS + sid) * _L
            ty1 = y1[pl.ds(off, _L)]
            tx1 = x1[pl.ds(off, _L)]
            ty2 = y2[pl.ds(off, _L)]
            tx2 = x2[pl.ds(off, _L)]
            tarea = (ty2 - ty1) * (tx2 - tx1)
            kv = keep[pl.ds(off, _L)]
            for p in range(_L):
                hh = jnp.maximum(jnp.minimum(pye[p], ty2) - jnp.maximum(pys[p], ty1), 0.0)
                ww = jnp.maximum(jnp.minimum(pxe[p], tx2) - jnp.maximum(pxs[p], tx1), 0.0)
                inter = hh * ww
                iou = inter / (pa[p] + tarea - inter + _EPS)
                kv = jnp.where(iou > pthr[p], 0.0, kv)
            keep[pl.ds(off, _L)] = kv
            return 0

        lax.fori_loop(jlo, _VPT, _apply, 0)
        return 0

    lax.fori_loop(0, _NB, _chunk, 0)

    for j in range(_VPT):
        off = (j * _NS + sid) * _L
        kv = keep[pl.ds(off, _L)]
        y1[pl.ds(off, _L)] = y1[pl.ds(off, _L)] * kv
        x1[pl.ds(off, _L)] = x1[pl.ds(off, _L)] * kv
        y2[pl.ds(off, _L)] = y2[pl.ds(off, _L)] * kv
        x2[pl.ds(off, _L)] = x2[pl.ds(off, _L)] * kv
        s[pl.ds(off, _L)] = s[pl.ds(off, _L)] * kv
        pltpu.sync_copy(y1.at[pl.ds(off, _L)], outh.at[0, pl.ds(off, _L)])
        pltpu.sync_copy(x1.at[pl.ds(off, _L)], outh.at[1, pl.ds(off, _L)])
        pltpu.sync_copy(y2.at[pl.ds(off, _L)], outh.at[2, pl.ds(off, _L)])
        pltpu.sync_copy(x2.at[pl.ds(off, _L)], outh.at[3, pl.ds(off, _L)])
        pltpu.sync_copy(s.at[pl.ds(off, _L)], outh.at[4, pl.ds(off, _L)])


_nms = functools.partial(
    pl.kernel,
    out_type=jax.ShapeDtypeStruct((5, _NPAD), jnp.float32),
    mesh=plsc.VectorSubcoreMesh(
        core_axis_name="c", subcore_axis_name="s", num_cores=1, num_subcores=_NS
    ),
    scratch_types=[
        pltpu.VMEM((_NPAD,), jnp.float32),
        pltpu.VMEM((_NPAD,), jnp.float32),
        pltpu.VMEM((_NPAD,), jnp.float32),
        pltpu.VMEM((_NPAD,), jnp.float32),
        pltpu.VMEM((_NPAD,), jnp.float32),
        pltpu.VMEM((_NPAD,), jnp.float32),
        pltpu.VMEM((_L,), jnp.float32),
        pltpu.VMEM_SHARED((_NPAD,), jnp.float32),
    ],
)(_nms_body)


def kernel(boxes, scores):
    n = boxes.shape[0]
    order = jnp.argsort(-scores)
    b = jnp.take(boxes, order, axis=0)
    s = jnp.take(scores, order, axis=0)
    pad = _NPAD - n
    y1 = jnp.pad(b[:, 0], (0, pad))
    x1 = jnp.pad(b[:, 1], (0, pad))
    y2 = jnp.pad(b[:, 2], (0, pad))
    x2 = jnp.pad(b[:, 3], (0, pad))
    sp = jnp.pad(s, (0, pad))
    out = _nms(y1, x1, y2, x2, sp)
    return out.T[:n]


# speedup vs baseline: 18.0717x; 18.0717x over previous
import functools

import jax
import jax.numpy as jnp
from jax import lax
from jax.experimental import pallas as pl
from jax.experimental.pallas import tpu as pltpu
from jax.experimental.pallas import tpu_sc as plsc

_L = 16
_NS = 16
_NPAD = 5120
_NB = _NPAD // _L
_VPT = _NB // _NS
_T = 0.3
_ST = 0.05
_BIG = 2e30
_EPS = 1e-9


def _bcast(v, l):
    idx = jnp.full((_L,), l, jnp.int32)
    return v.at[idx].get(mode="promise_in_bounds")


def _nms_body(y1h, x1h, y2h, x2h, sh, outh, y1, x1, y2, x2, s, keep, slot, shared):
    sid = lax.axis_index("s")

    pltpu.sync_copy(y1h, y1)
    pltpu.sync_copy(x1h, x1)
    pltpu.sync_copy(y2h, y2)
    pltpu.sync_copy(x2h, x2)
    pltpu.sync_copy(sh, s)

    def _init(j, _):
        off = (j * _NS + sid) * _L
        sv = s[pl.ds(off, _L)]
        keep[pl.ds(off, _L)] = jnp.where(sv > _ST, 1.0, 0.0)
        return 0

    lax.fori_loop(0, _VPT, _init, 0)

    lane = lax.iota(jnp.int32, _L)

    def _chunk(k, _):
        owner = lax.rem(k, _NS)
        base = k * _L

        @pl.when(sid == owner)
        def _():
            py1 = y1[pl.ds(base, _L)]
            px1 = x1[pl.ds(base, _L)]
            py2 = y2[pl.ds(base, _L)]
            px2 = x2[pl.ds(base, _L)]
            parea = (py2 - py1) * (px2 - px1)
            kv = keep[pl.ds(base, _L)]
            for l in range(_L):
                ay1 = _bcast(py1, l)
                ax1 = _bcast(px1, l)
                ay2 = _bcast(py2, l)
                ax2 = _bcast(px2, l)
                aarea = (ay2 - ay1) * (ax2 - ax1)
                klv = _bcast(kv, l)
                hh = jnp.maximum(jnp.minimum(ay2, py2) - jnp.maximum(ay1, py1), 0.0)
                ww = jnp.maximum(jnp.minimum(ax2, px2) - jnp.maximum(ax1, px1), 0.0)
                inter = hh * ww
                iou = inter / (aarea + parea - inter + _EPS)
                thr = jnp.where(klv > 0.0, jnp.float32(_T), jnp.float32(_BIG))
                sup = (iou > thr) & (lane > l)
                kv = jnp.where(sup, 0.0, kv)
            keep[pl.ds(base, _L)] = kv
            pltpu.sync_copy(keep.at[pl.ds(base, _L)], shared.at[pl.ds(base, _L)])

        plsc.subcore_barrier()

        pltpu.sync_copy(shared.at[pl.ds(base, _L)], slot)
        slotv = slot[...]

        my1 = y1[pl.ds(base, _L)] * slotv
        mx1 = x1[pl.ds(base, _L)] * slotv
        my2 = y2[pl.ds(base, _L)] * slotv
        mx2 = x2[pl.ds(base, _L)] * slotv
        bs = [
            (_bcast(my1, p), _bcast(mx1, p), _bcast(my2, p), _bcast(mx2, p))
            for p in range(_L)
        ]
        pa = [(b[2] - b[0]) * (b[3] - b[1]) for b in bs]

        jlo = lax.div(k - sid + _NS, _NS)

        def _apply(j, _):
            off = (j * _NS + sid) * _L
            ty1 = y1[pl.ds(off, _L)]
            tx1 = x1[pl.ds(off, _L)]
            ty2 = y2[pl.ds(off, _L)]
            tx2 = x2[pl.ds(off, _L)]
            tarea = (ty2 - ty1) * (tx2 - tx1)
            kv = keep[pl.ds(off, _L)]
            for p in range(_L):
                by1, bx1, by2, bx2 = bs[p]
                hh = jnp.maximum(jnp.minimum(by2, ty2) - jnp.maximum(by1, ty1), 0.0)
                ww = jnp.maximum(jnp.minimum(bx2, tx2) - jnp.maximum(bx1, tx1), 0.0)
                inter = hh * ww
                iou = inter / (pa[p] + tarea - inter + _EPS)
                kv = jnp.where(iou > jnp.float32(_T), 0.0, kv)
            keep[pl.ds(off, _L)] = kv
            return 0

        lax.fori_loop(jlo, _VPT, _apply, 0)

        return 0

    lax.fori_loop(0, _NB, _chunk, 0)

    def _out(j, _):
        off = (j * _NS + sid) * _L
        kv = keep[pl.ds(off, _L)]
        y1[pl.ds(off, _L)] = y1[pl.ds(off, _L)] * kv
        x1[pl.ds(off, _L)] = x1[pl.ds(off, _L)] * kv
        y2[pl.ds(off, _L)] = y2[pl.ds(off, _L)] * kv
        x2[pl.ds(off, _L)] = x2[pl.ds(off, _L)] * kv
        s[pl.ds(off, _L)] = s[pl.ds(off, _L)] * kv
        pltpu.sync_copy(y1.at[pl.ds(off, _L)], outh.at[0, pl.ds(off, _L)])
        pltpu.sync_copy(x1.at[pl.ds(off, _L)], outh.at[1, pl.ds(off, _L)])
        pltpu.sync_copy(y2.at[pl.ds(off, _L)], outh.at[2, pl.ds(off, _L)])
        pltpu.sync_copy(x2.at[pl.ds(off, _L)], outh.at[3, pl.ds(off, _L)])
        pltpu.sync_copy(s.at[pl.ds(off, _L)], outh.at[4, pl.ds(off, _L)])
        return 0

    lax.fori_loop(0, _VPT, _out, 0)


_nms = functools.partial(
    pl.kernel,
    out_type=jax.ShapeDtypeStruct((5, _NPAD), jnp.float32),
    mesh=plsc.VectorSubcoreMesh(
        core_axis_name="c", subcore_axis_name="s", num_cores=1, num_subcores=_NS
    ),
    scratch_types=[
        pltpu.VMEM((_NPAD,), jnp.float32),
        pltpu.VMEM((_NPAD,), jnp.float32),
        pltpu.VMEM((_NPAD,), jnp.float32),
        pltpu.VMEM((_NPAD,), jnp.float32),
        pltpu.VMEM((_NPAD,), jnp.float32),
        pltpu.VMEM((_NPAD,), jnp.float32),
        pltpu.VMEM((_L,), jnp.float32),
        pltpu.VMEM_SHARED((_NPAD,), jnp.float32),
    ],
)(_nms_body)


def kernel(boxes, scores):
    n = boxes.shape[0]
    order = jnp.argsort(-scores)
    b = jnp.take(boxes, order, axis=0)
    s = jnp.take(scores, order, axis=0)
    pad = _NPAD - n
    y1 = jnp.pad(b[:, 0], (0, pad))
    x1 = jnp.pad(b[:, 1], (0, pad))
    y2 = jnp.pad(b[:, 2], (0, pad))
    x2 = jnp.pad(b[:, 3], (0, pad))
    sp = jnp.pad(s, (0, pad))
    out = _nms(y1, x1, y2, x2, sp)
    return out.T[:n]
